# baseline (speedup 1.0000x reference)
import jax
import jax.numpy as jnp
from jax import lax
from jax.experimental import pallas as pl
from jax.experimental.pallas import tpu as pltpu
from jax.experimental.pallas import tpu_sc as plsc

HIDDEN = 64
MARGIN = 1.0
BATCH = 16384
NROW = 1000000

L = 16
NC = 2
NS = 16
NW = NC * NS
BPW = BATCH // NW
CHUNK = 16
NCHUNK = BPW // CHUNK
IPW = 6 * BPW


def _score_chunk(idx_v, off, sbuf, rbuf, obuf, score_v, score_off):
    for g in range(CHUNK // L):
        rowid = lax.iota(jnp.int32, L) + g * L
        ssub = idx_v[pl.ds(off + 0 * CHUNK + g * L, L)] & 7
        rsub = idx_v[pl.ds(off + 1 * CHUNK + g * L, L)] & 7
        osub = idx_v[pl.ds(off + 2 * CHUNK + g * L, L)] & 7

        def dbody(j, acc):
            for u in range(4):
                d = jnp.full((L,), j * 4 + u, jnp.int32)
                s = plsc.load_gather(sbuf, [rowid, ssub, d])
                r = plsc.load_gather(rbuf, [rowid, rsub, d])
                o = plsc.load_gather(obuf, [rowid, osub, d])
                acc = acc + jnp.abs(s + r - o)
            return acc

        acc = lax.fori_loop(0, HIDDEN // 4, dbody, jnp.zeros((L,), jnp.float32))
        score_v[pl.ds(score_off + g * L, L)] = acc


def _tec_body(idx_hbm, ent_hbm, rel_hbm, out_hbm,
              idx_v, sbuf0, rbuf0, obuf0, sbuf1, rbuf1, obuf1,
              score_v, out_v, sem0, sem1):
    wid = lax.axis_index("s") * NC + lax.axis_index("c")
    base = wid * BPW

    ent2 = ent_hbm.at[pl.ds(0, NROW)].reshape(NROW // 8, 8, HIDDEN)
    rel2 = rel_hbm.at[pl.ds(0, NROW)].reshape(NROW // 8, 8, HIDDEN)

    pltpu.sync_copy(idx_hbm.at[pl.ds(wid * IPW, IPW)], idx_v)

    def issue_chunk(p, sb, rb, ob, sem):
        off = p * (3 * CHUNK)
        sv = lax.shift_right_logical(idx_v[pl.ds(off + 0 * CHUNK, L)], 3)
        rv = lax.shift_right_logical(idx_v[pl.ds(off + 1 * CHUNK, L)], 3)
        ov = lax.shift_right_logical(idx_v[pl.ds(off + 2 * CHUNK, L)], 3)
        for j in range(L):
            pltpu.async_copy(ent2.at[sv[j]], sb.at[j], sem)
            pltpu.async_copy(rel2.at[rv[j]], rb.at[j], sem)
            pltpu.async_copy(ent2.at[ov[j]], ob.at[j], sem)

    def drain_chunk(sb, rb, ob, sem):
        pltpu.make_async_copy(ent2.at[pl.ds(0, CHUNK)], sb, sem).wait()
        pltpu.make_async_copy(rel2.at[pl.ds(0, CHUNK)], rb, sem).wait()
        pltpu.make_async_copy(ent2.at[pl.ds(0, CHUNK)], ob, sem).wait()

    NPAIR = NCHUNK
    issue_chunk(0, sbuf0, rbuf0, obuf0, sem0)

    def pair_body(q, _):
        issue_chunk(2 * q + 1, sbuf1, rbuf1, obuf1, sem1)
        drain_chunk(sbuf0, rbuf0, obuf0, sem0)
        _score_chunk(idx_v, (2 * q) * (3 * CHUNK), sbuf0, rbuf0, obuf0,
                     score_v, (2 * q) * CHUNK)

        @pl.when(q < NPAIR - 1)
        def _():
            issue_chunk(2 * q + 2, sbuf0, rbuf0, obuf0, sem0)

        drain_chunk(sbuf1, rbuf1, obuf1, sem1)
        _score_chunk(idx_v, (2 * q + 1) * (3 * CHUNK), sbuf1, rbuf1, obuf1,
                     score_v, (2 * q + 1) * CHUNK)
        return 0

    lax.fori_loop(0, NPAIR, pair_body, 0)

    def loss_body(k, _):
        p = score_v[pl.ds(k * L, L)]
        n = score_v[pl.ds(BPW + k * L, L)]
        out_v[pl.ds(k * L, L)] = jnp.maximum(p - n + MARGIN, 0.0)
        return 0

    lax.fori_loop(0, BPW // L, loss_body, 0)
    pltpu.sync_copy(out_v, out_hbm.at[pl.ds(base, BPW)])


@jax.jit
def _transe_sc(idx, entity_embedding, relation_embedding):
    run = pl.kernel(
        _tec_body,
        out_type=jax.ShapeDtypeStruct((BATCH,), jnp.float32),
        mesh=plsc.VectorSubcoreMesh(core_axis_name="c", subcore_axis_name="s"),
        compiler_params=pltpu.CompilerParams(
            needs_layout_passes=False, skip_device_barrier=True),
        scratch_types=[
            pltpu.VMEM((IPW,), jnp.int32),
            pltpu.VMEM((CHUNK, 8, HIDDEN), jnp.float32),
            pltpu.VMEM((CHUNK, 8, HIDDEN), jnp.float32),
            pltpu.VMEM((CHUNK, 8, HIDDEN), jnp.float32),
            pltpu.VMEM((CHUNK, 8, HIDDEN), jnp.float32),
            pltpu.VMEM((CHUNK, 8, HIDDEN), jnp.float32),
            pltpu.VMEM((CHUNK, 8, HIDDEN), jnp.float32),
            pltpu.VMEM((2 * BPW,), jnp.float32),
            pltpu.VMEM((BPW,), jnp.float32),
            pltpu.SemaphoreType.DMA,
            pltpu.SemaphoreType.DMA,
        ],
    )
    return run(idx, entity_embedding, relation_embedding)


def kernel(positive_triple, negative_triple, entity_embedding, relation_embedding):
    pos = positive_triple.astype(jnp.int32)
    neg = negative_triple.astype(jnp.int32)
    idx6 = jnp.stack(
        [pos[:, 0], pos[:, 1], pos[:, 2], neg[:, 0], neg[:, 1], neg[:, 2]], axis=0)
    idx = jnp.transpose(
        idx6.reshape(2, 3, NW, NCHUNK, CHUNK), (2, 0, 3, 1, 4)).reshape(-1)
    return _transe_sc(idx, entity_embedding, relation_embedding)
